# Initial kernel scaffold; baseline (speedup 1.0000x reference)
import functools

import jax
import jax.numpy as jnp
from jax import lax
from jax.experimental import pallas as pl
from jax.experimental.pallas import tpu as pltpu
from jax.experimental.pallas import tpu_sc as plsc

NC = 2
NS = 16
NW = NC * NS
LANES = 16


def _sc_sum_pool(x_r, emb_table, B, H, D, spw, chunk):
    h2 = H // 2
    nchunks = spw // chunk
    mesh = plsc.VectorSubcoreMesh(
        core_axis_name="c", subcore_axis_name="s",
        num_cores=NC, num_subcores=NS)

    @functools.partial(
        pl.kernel,
        out_type=jax.ShapeDtypeStruct((B, D), jnp.float32),
        mesh=mesh,
        scratch_types=[
            pltpu.VMEM((chunk, 2, h2), jnp.int32),
            pltpu.VMEM((chunk, 2, h2), jnp.int32),
            pltpu.VMEM((2, h2, D), jnp.float32),
            pltpu.VMEM((2, h2, D), jnp.float32),
            pltpu.VMEM((spw, D), jnp.float32),
            pltpu.SemaphoreType.DMA,
            pltpu.SemaphoreType.DMA,
            pltpu.SemaphoreType.DMA,
        ],
    )
    def body(x_hbm, emb_hbm, out_hbm, ibuf0, ibuf1, rbuf0, rbuf1,
             pooled, semi, sem0, sem1):
        wid = lax.axis_index("s") * NC + lax.axis_index("c")
        base = wid * spw
        ibufs = (ibuf0, ibuf1)
        rbufs = (rbuf0, rbuf1)
        sems = (sem0, sem1)

        def idx_copy(c):
            pltpu.make_async_copy(
                x_hbm.at[pl.ds(base + c * chunk, chunk)],
                ibufs[c % 2], semi).start()

        def idx_wait(c):
            pltpu.make_async_copy(
                x_hbm.at[pl.ds(base + c * chunk, chunk)],
                ibufs[c % 2], semi).wait()

        def fire(ib, i, rbi):
            for j in range(2):
                pltpu.make_async_copy(
                    emb_hbm.at[ibufs[ib].at[i, j]],
                    rbufs[rbi].at[j], sems[rbi]).start()

        def drain(ib, i, rbi):
            for j in range(2):
                pltpu.make_async_copy(
                    emb_hbm.at[ibufs[ib].at[i, j]],
                    rbufs[rbi].at[j], sems[rbi]).wait()

        def reduce(rbi, sl):
            rb = rbufs[rbi]
            zero = jnp.zeros((LANES,), jnp.float32)

            def rbody(r2, accs):
                a0, a1, a2, a3, a4, a5, a6, a7 = accs
                r = 2 * r2
                a0 = a0 + rb[0, r, pl.ds(0, LANES)]
                a1 = a1 + rb[0, r, pl.ds(LANES, LANES)]
                a2 = a2 + rb[1, r, pl.ds(0, LANES)]
                a3 = a3 + rb[1, r, pl.ds(LANES, LANES)]
                a4 = a4 + rb[0, r + 1, pl.ds(0, LANES)]
                a5 = a5 + rb[0, r + 1, pl.ds(LANES, LANES)]
                a6 = a6 + rb[1, r + 1, pl.ds(0, LANES)]
                a7 = a7 + rb[1, r + 1, pl.ds(LANES, LANES)]
                return (a0, a1, a2, a3, a4, a5, a6, a7)

            a = lax.fori_loop(0, h2 // 2, rbody, (zero,) * 8)
            lo = (a[0] + a[2]) + (a[4] + a[6])
            hi = (a[1] + a[3]) + (a[5] + a[7])
            pooled[sl, pl.ds(0, LANES)] = lo
            pooled[sl, pl.ds(LANES, LANES)] = hi

        idx_copy(0)
        idx_wait(0)
        for c in range(nchunks):
            ib = c % 2
            if c + 1 < nchunks:
                idx_copy(c + 1)
            fire(ib, 0, 0)
            fire(ib, 1, 1)

            def pbody(p, _, ib=ib, c=c):
                i0 = 2 * p
                last = chunk - 1
                drain(ib, i0, 0)
                reduce(0, c * chunk + i0)
                fire(ib, jnp.minimum(i0 + 2, last), 0)
                drain(ib, i0 + 1, 1)
                reduce(1, c * chunk + i0 + 1)
                fire(ib, jnp.minimum(i0 + 3, last), 1)
                return 0

            lax.fori_loop(0, chunk // 2, pbody, 0)
            drain(ib, chunk - 1, 0)
            drain(ib, chunk - 1, 1)
            if c + 1 < nchunks:
                idx_wait(c + 1)

        pltpu.sync_copy(pooled, out_hbm.at[pl.ds(base, spw)])

    return body(x_r, emb_table)


def _tc_linear_relu(sums, fc_w, fc_b2, inv_h, B, D):
    nblk = 8
    blk = B // nblk

    def body(s_ref, w_ref, b_ref, o_ref):
        pooled = s_ref[...] * inv_h
        acc = lax.dot_general(
            pooled, w_ref[...], (((1,), (1,)), ((), ())),
            preferred_element_type=jnp.float32)
        o_ref[...] = jnp.maximum(acc + b_ref[...], 0.0)

    return pl.pallas_call(
        body,
        out_shape=jax.ShapeDtypeStruct((B, D), jnp.float32),
        grid=(nblk,),
        in_specs=[
            pl.BlockSpec((blk, D), lambda i: (i, 0)),
            pl.BlockSpec((D, D), lambda i: (0, 0)),
            pl.BlockSpec((1, D), lambda i: (0, 0)),
        ],
        out_specs=pl.BlockSpec((blk, D), lambda i: (i, 0)),
    )(sums, fc_w, fc_b2)


def kernel(x, emb_table, fc_w, fc_b):
    B, H = x.shape
    D = emb_table.shape[1]
    assert B % NW == 0 and H % 2 == 0 and H // 2 <= 128 and D == 2 * LANES
    spw = B // NW
    chunk = 128
    assert spw % chunk == 0 and chunk % 2 == 0

    x_r = x.astype(jnp.int32).reshape(B, 2, H // 2)
    sums = _sc_sum_pool(x_r, emb_table, B, H, D, spw, chunk)
    return _tc_linear_relu(sums, fc_w, fc_b.reshape(1, D), 1.0 / H, B, D)


# speedup vs baseline: 13.6055x; 13.6055x over previous
import functools

import jax
import jax.numpy as jnp
from jax import lax
from jax.experimental import pallas as pl
from jax.experimental.pallas import tpu as pltpu
from jax.experimental.pallas import tpu_sc as plsc

NC = 2
NS = 16
NW = NC * NS
LANES = 16


def _sc_sum_pool(x_r, emb_table, B, H, D, spw, chunk):
    h2 = H // 2
    nchunks = spw // chunk
    mesh = plsc.VectorSubcoreMesh(
        core_axis_name="c", subcore_axis_name="s",
        num_cores=NC, num_subcores=NS)

    @functools.partial(
        pl.kernel,
        out_type=jax.ShapeDtypeStruct((B, D), jnp.float32),
        mesh=mesh,
        compiler_params=pltpu.CompilerParams(use_tc_tiling_on_sc=False),
        scratch_types=[
            pltpu.VMEM((chunk, 2, h2), jnp.int32),
            pltpu.VMEM((chunk, 2, h2), jnp.int32),
            pltpu.VMEM((2, h2, D), jnp.float32),
            pltpu.VMEM((2, h2, D), jnp.float32),
            pltpu.VMEM((spw, D), jnp.float32),
            pltpu.SemaphoreType.DMA,
            pltpu.SemaphoreType.DMA,
            pltpu.SemaphoreType.DMA,
        ],
    )
    def body(x_hbm, emb_hbm, out_hbm, ibuf0, ibuf1, rbuf0, rbuf1,
             pooled, semi, sem0, sem1):
        wid = lax.axis_index("s") * NC + lax.axis_index("c")
        base = wid * spw
        ibufs = (ibuf0, ibuf1)
        rbufs = (rbuf0, rbuf1)
        sems = (sem0, sem1)

        def idx_copy(c):
            pltpu.make_async_copy(
                x_hbm.at[pl.ds(base + c * chunk, chunk)],
                ibufs[c % 2], semi).start()

        def idx_wait(c):
            pltpu.make_async_copy(
                x_hbm.at[pl.ds(base + c * chunk, chunk)],
                ibufs[c % 2], semi).wait()

        def fire(ib, i, rbi):
            for j in range(2):
                pltpu.make_async_copy(
                    emb_hbm.at[ibufs[ib].at[i, j]],
                    rbufs[rbi].at[j], sems[rbi]).start()

        def drain(ib, i, rbi):
            for j in range(2):
                pltpu.make_async_copy(
                    emb_hbm.at[ibufs[ib].at[i, j]],
                    rbufs[rbi].at[j], sems[rbi]).wait()

        def reduce(rbi, sl):
            rb = rbufs[rbi]
            zero = jnp.zeros((LANES,), jnp.float32)

            def rbody(r2, accs):
                a0, a1, a2, a3, a4, a5, a6, a7 = accs
                r = 2 * r2
                a0 = a0 + rb[0, r, pl.ds(0, LANES)]
                a1 = a1 + rb[0, r, pl.ds(LANES, LANES)]
                a2 = a2 + rb[1, r, pl.ds(0, LANES)]
                a3 = a3 + rb[1, r, pl.ds(LANES, LANES)]
                a4 = a4 + rb[0, r + 1, pl.ds(0, LANES)]
                a5 = a5 + rb[0, r + 1, pl.ds(LANES, LANES)]
                a6 = a6 + rb[1, r + 1, pl.ds(0, LANES)]
                a7 = a7 + rb[1, r + 1, pl.ds(LANES, LANES)]
                return (a0, a1, a2, a3, a4, a5, a6, a7)

            a = lax.fori_loop(0, h2 // 2, rbody, (zero,) * 8)
            lo = (a[0] + a[2]) + (a[4] + a[6])
            hi = (a[1] + a[3]) + (a[5] + a[7])
            pooled[sl, pl.ds(0, LANES)] = lo
            pooled[sl, pl.ds(LANES, LANES)] = hi

        idx_copy(0)
        idx_wait(0)
        for c in range(nchunks):
            ib = c % 2
            if c + 1 < nchunks:
                idx_copy(c + 1)
            fire(ib, 0, 0)
            fire(ib, 1, 1)

            def pbody(p, _, ib=ib, c=c):
                i0 = 2 * p
                last = chunk - 1
                drain(ib, i0, 0)
                reduce(0, c * chunk + i0)
                fire(ib, jnp.minimum(i0 + 2, last), 0)
                drain(ib, i0 + 1, 1)
                reduce(1, c * chunk + i0 + 1)
                fire(ib, jnp.minimum(i0 + 3, last), 1)
                return 0

            lax.fori_loop(0, chunk // 2, pbody, 0)
            drain(ib, chunk - 1, 0)
            drain(ib, chunk - 1, 1)
            if c + 1 < nchunks:
                idx_wait(c + 1)

        pltpu.sync_copy(pooled, out_hbm.at[pl.ds(base, spw)])

    return body(x_r, emb_table)


def _tc_linear_relu(sums, fc_w, fc_b2, inv_h, B, D):
    nblk = 8
    blk = B // nblk

    def body(s_ref, w_ref, b_ref, o_ref):
        pooled = s_ref[...] * inv_h
        acc = lax.dot_general(
            pooled, w_ref[...], (((1,), (1,)), ((), ())),
            preferred_element_type=jnp.float32)
        o_ref[...] = jnp.maximum(acc + b_ref[...], 0.0)

    return pl.pallas_call(
        body,
        out_shape=jax.ShapeDtypeStruct((B, D), jnp.float32),
        grid=(nblk,),
        in_specs=[
            pl.BlockSpec((blk, D), lambda i: (i, 0)),
            pl.BlockSpec((D, D), lambda i: (0, 0)),
            pl.BlockSpec((1, D), lambda i: (0, 0)),
        ],
        out_specs=pl.BlockSpec((blk, D), lambda i: (i, 0)),
    )(sums, fc_w, fc_b2)


def kernel(x, emb_table, fc_w, fc_b):
    B, H = x.shape
    D = emb_table.shape[1]
    assert B % NW == 0 and H % 2 == 0 and H // 2 <= 128 and D == 2 * LANES
    spw = B // NW
    chunk = 128
    assert spw % chunk == 0 and chunk % 2 == 0

    x_r = x.astype(jnp.int32).reshape(B, 2, H // 2)
    sums = _sc_sum_pool(x_r, emb_table, B, H, D, spw, chunk)
    return _tc_linear_relu(sums, fc_w, fc_b.reshape(1, D), 1.0 / H, B, D)
